# Initial kernel scaffold; baseline (speedup 1.0000x reference)
import jax
import jax.numpy as jnp
from jax.experimental import pallas as pl


def kernel(anchors, rpn_bbox_pred, scores):
    raise NotImplementedError("write your pallas kernel here")


# speedup vs baseline: 27.7415x; 27.7415x over previous
import jax
import jax.numpy as jnp
from jax.experimental import pallas as pl

_N = 20000
_MAX_OUT = 128
_IOU = 0.7
_IM = 224.0
_NP = 20480
_R = _NP // 128


def _nms_body(a_ref, d_ref, s_ref, o_ref):
    x1 = a_ref[0]
    y1 = a_ref[1]
    z1 = a_ref[2]
    x2 = a_ref[3]
    y2 = a_ref[4]
    z2 = a_ref[5]
    dx = d_ref[0]
    dy = d_ref[1]
    dz = d_ref[2]
    dw = d_ref[3]
    dh = d_ref[4]
    dl = d_ref[5]

    w = x2 - x1 + 1.0
    h = y2 - y1 + 1.0
    l = z2 - z1 + 1.0
    cx = x1 + w * 0.5
    cy = y1 + h * 0.5
    cz = z1 + l * 0.5
    pcx = dx * w + cx
    pcy = dy * h + cy
    pcz = dz * l + cz
    pw = jnp.exp(dw) * w
    ph = jnp.exp(dh) * h
    pl_ = jnp.exp(dl) * l

    hi = _IM - 1.0
    bx1 = jnp.clip(pcx - pw * 0.5, 0.0, hi)
    by1 = jnp.clip(pcy - ph * 0.5, 0.0, hi)
    bz1 = jnp.clip(pcz - pl_ * 0.5, 0.0, hi)
    bx2 = jnp.clip(pcx + pw * 0.5, 0.0, hi)
    by2 = jnp.clip(pcy + ph * 0.5, 0.0, hi)
    bz2 = jnp.clip(pcz + pl_ * 0.5, 0.0, hi)

    vols = (bx2 - bx1 + 1.0) * (by2 - by1 + 1.0) * (bz2 - bz1 + 1.0)

    rows = jax.lax.broadcasted_iota(jnp.int32, (_R, 128), 0)
    cols = jax.lax.broadcasted_iota(jnp.int32, (_R, 128), 1)
    fidx = rows * 128 + cols
    scores = s_ref[...]
    avail0 = jnp.where(fidx < _N, 1.0, 0.0).astype(jnp.float32)
    lane = jax.lax.broadcasted_iota(jnp.int32, (1, 128), 1)

    def body(t, avail_f):
        ms = jnp.where(avail_f > 0.0, scores, -1.0)
        m = jnp.max(ms)
        valid_f = jnp.where(m >= 0.0, 1.0, 0.0).astype(jnp.float32)
        idx = jnp.min(jnp.where(ms == m, fidx, _NP))
        sel = fidx == idx
        sx1 = jnp.sum(jnp.where(sel, bx1, 0.0))
        sy1 = jnp.sum(jnp.where(sel, by1, 0.0))
        sz1 = jnp.sum(jnp.where(sel, bz1, 0.0))
        sx2 = jnp.sum(jnp.where(sel, bx2, 0.0))
        sy2 = jnp.sum(jnp.where(sel, by2, 0.0))
        sz2 = jnp.sum(jnp.where(sel, bz2, 0.0))
        ssc = jnp.sum(jnp.where(sel, scores, 0.0))

        xx1 = jnp.maximum(sx1, bx1)
        yy1 = jnp.maximum(sy1, by1)
        zz1 = jnp.maximum(sz1, bz1)
        xx2 = jnp.minimum(sx2, bx2)
        yy2 = jnp.minimum(sy2, by2)
        zz2 = jnp.minimum(sz2, bz2)
        inter = (jnp.maximum(xx2 - xx1 + 1.0, 0.0)
                 * jnp.maximum(yy2 - yy1 + 1.0, 0.0)
                 * jnp.maximum(zz2 - zz1 + 1.0, 0.0))
        vol0 = (sx2 - sx1 + 1.0) * (sy2 - sy1 + 1.0) * (sz2 - sz1 + 1.0)
        iou = inter / (vol0 + vols - inter)
        supp_f = jnp.where(iou >= _IOU, 1.0, 0.0).astype(jnp.float32)
        sel_f = jnp.where(sel, 1.0, 0.0).astype(jnp.float32)

        new_avail_f = avail_f * (1.0 - valid_f * supp_f) * (1.0 - sel_f)

        row = jnp.zeros((1, 128), jnp.float32)
        for c, v in enumerate((sx1, sy1, sz1, sx2, sy2, sz2, ssc)):
            row = jnp.where(lane == c, v, row)
        o_ref[pl.ds(t, 1), :] = row * valid_f
        return new_avail_f

    jax.lax.fori_loop(0, _MAX_OUT, body, avail0)


def kernel(anchors, rpn_bbox_pred, scores):
    pad = _NP - _N
    a = jnp.pad(anchors, ((0, pad), (0, 0))).T.reshape(6, _R, 128)
    d = jnp.pad(rpn_bbox_pred, ((0, pad), (0, 0))).T.reshape(6, _R, 128)
    s = jnp.pad(scores, (0, pad)).reshape(_R, 128)
    out = pl.pallas_call(
        _nms_body,
        out_shape=jax.ShapeDtypeStruct((_MAX_OUT, 128), jnp.float32),
    )(a, d, s)
    return out[:, :7]
